# baseline (speedup 1.0000x reference)
import functools

import jax
import jax.numpy as jnp
from jax import lax
from jax.experimental import pallas as pl
from jax.experimental.pallas import tpu as pltpu
from jax.experimental.pallas import tpu_sc as plsc

VOC = 100000
D = 128
T = 100
B = 16384
NC = 2
NS = 16
NW = NC * NS
BH = B // 2
SEQ_PER_W = BH // NW
S_CH = 2
N_CH = SEQ_PER_W // S_CH
NSLOT = 4


def _body(tok_hbm, pos_hbm, idx_hbm, out_hbm, idx_v, rows_v, *sems):
    sem_ab = sems[0:NSLOT]
    sem_c = sems[NSLOT:2 * NSLOT]
    sem_d = sems[2 * NSLOT:3 * NSLOT]
    wid = lax.axis_index("s") * NC + lax.axis_index("c")
    wbase = wid * SEQ_PER_W

    def ab_copies(c, s):
        seq0 = wbase + c * S_CH
        ops = [pltpu.make_async_copy(
            idx_hbm.at[pl.ds(seq0, S_CH)], idx_v.at[s], sem_ab[s])]
        for j in range(S_CH):
            ops.append(pltpu.make_async_copy(
                pos_hbm.at[wid], rows_v.at[s].at[j], sem_ab[s]))
        return ops

    def c_copies(s):
        return [pltpu.make_async_copy(
            tok_hbm.at[idx_v.at[s].at[j]], rows_v.at[s].at[j], sem_c[s])
            for j in range(S_CH)]

    def d_copy(c, s):
        seq0 = wbase + c * S_CH
        return pltpu.make_async_copy(
            rows_v.at[s], out_hbm.at[pl.ds(seq0, S_CH)], sem_d[s])

    def group(g, carry):
        for k in range(NSLOT):
            c = g * NSLOT + k

            @pl.when(jnp.logical_and(c >= NSLOT, c < N_CH + NSLOT))
            def _():
                d_copy(c - NSLOT, k).wait()

            @pl.when(c < N_CH)
            def _():
                for op in ab_copies(c, k):
                    op.start()

            @pl.when(jnp.logical_and(c >= 1, c < N_CH + 1))
            def _():
                kk = (k - 1) % NSLOT
                for op in ab_copies(c - 1, kk):
                    op.wait()
                for op in c_copies(kk):
                    op.start(add=True)

            @pl.when(jnp.logical_and(c >= 2, c < N_CH + 2))
            def _():
                kk = (k - 2) % NSLOT
                for op in c_copies(kk):
                    op.wait()
                d_copy(c - 2, kk).start()
        return carry

    lax.fori_loop(0, (N_CH + NSLOT) // NSLOT, group, 0)


@jax.jit
def _emb(tok_table, pos_table, idx):
    grid_kernel = pl.kernel(
        _body,
        out_type=jax.ShapeDtypeStruct((BH, T, D), jnp.float32),
        mesh=plsc.VectorSubcoreMesh(
            core_axis_name="c", subcore_axis_name="s",
            num_cores=NC, num_subcores=NS),
        scratch_types=[
            pltpu.VMEM((NSLOT, S_CH, T), jnp.int32),
            pltpu.VMEM((NSLOT, S_CH, T, D), jnp.float32),
        ] + [pltpu.SemaphoreType.DMA] * (3 * NSLOT),
    )
    return grid_kernel(tok_table, pos_table, idx)


def kernel(inpTok, tok_table, pos_table):
    pos_rep = jnp.broadcast_to(pos_table, (NW, T, D))
    idx = inpTok.astype(jnp.int32)
    o1 = _emb(tok_table, pos_rep, idx[:BH])
    o2 = _emb(tok_table, pos_rep, idx[BH:])
    return jnp.concatenate([o1, o2], axis=0)


# speedup vs baseline: 1.6529x; 1.6529x over previous
import functools

import jax
import jax.numpy as jnp
from jax import lax
from jax.experimental import pallas as pl
from jax.experimental.pallas import tpu as pltpu
from jax.experimental.pallas import tpu_sc as plsc

VOC = 100000
D = 128
T = 100
B = 16384
NC = 2
NS = 16
NW = NC * NS
SEQ_PER_W = B // NW
S_CH = 2
N_CH = SEQ_PER_W // S_CH
NSLOT = 4


def _body(tok_hbm, pos_hbm, idx_hbm, out_hbm, idx_v, rows_v, pos_v, *sems):
    sem_ab = sems[0:NSLOT]
    sem_c = sems[NSLOT:2 * NSLOT]
    sem_d = sems[2 * NSLOT:3 * NSLOT]
    wid = lax.axis_index("s") * NC + lax.axis_index("c")
    wbase = wid * SEQ_PER_W

    pltpu.sync_copy(pos_hbm.at[wid], pos_v)

    def ab_copies(c, s):
        seq0 = wbase + c * S_CH
        ops = [pltpu.make_async_copy(
            idx_hbm.at[pl.ds(seq0, S_CH)], idx_v.at[s], sem_ab[s])]
        ops.append(pltpu.make_async_copy(
            pos_hbm.at[wid], rows_v.at[s].at[0], sem_ab[s]))
        return ops

    def add_pos(s):
        def row(r, carry):
            for v in range(8):
                sl = pl.ds(v * 16, 16)
                rows_v[s, 1, r, sl] = rows_v[s, 1, r, sl] + pos_v[r, sl]
            return carry
        lax.fori_loop(0, T, row, 0)

    def c_copies(s):
        return [pltpu.make_async_copy(
            tok_hbm.at[idx_v.at[s].at[j]], rows_v.at[s].at[j], sem_c[s])
            for j in range(S_CH)]

    def d_copy(c, s):
        seq0 = wbase + c * S_CH
        return pltpu.make_async_copy(
            rows_v.at[s], out_hbm.at[pl.ds(seq0, S_CH)], sem_d[s])

    def group(g, carry):
        for k in range(NSLOT):
            c = g * NSLOT + k

            @pl.when(jnp.logical_and(c >= NSLOT, c < N_CH + NSLOT))
            def _():
                d_copy(c - NSLOT, k).wait()

            @pl.when(c < N_CH)
            def _():
                for op in ab_copies(c, k):
                    op.start()

            @pl.when(jnp.logical_and(c >= 1, c < N_CH + 1))
            def _():
                kk = (k - 1) % NSLOT
                for op in ab_copies(c - 1, kk):
                    op.wait()
                ops = c_copies(kk)
                ops[0].start(add=True)
                ops[1].start()

            @pl.when(jnp.logical_and(c >= 2, c < N_CH + 2))
            def _():
                kk = (k - 2) % NSLOT
                for op in c_copies(kk):
                    op.wait()
                add_pos(kk)
                d_copy(c - 2, kk).start()
        return carry

    lax.fori_loop(0, (N_CH + NSLOT) // NSLOT, group, 0)


@jax.jit
def _emb(tok_table, pos_table, idx):
    grid_kernel = pl.kernel(
        _body,
        out_type=jax.ShapeDtypeStruct((B, T, D), jnp.float32),
        mesh=plsc.VectorSubcoreMesh(
            core_axis_name="c", subcore_axis_name="s",
            num_cores=NC, num_subcores=NS),
        scratch_types=[
            pltpu.VMEM((NSLOT, S_CH, T), jnp.int32),
            pltpu.VMEM((NSLOT, S_CH, T, D), jnp.float32),
            pltpu.VMEM((T, D), jnp.float32),
        ] + [pltpu.SemaphoreType.DMA] * (3 * NSLOT),
    )
    return grid_kernel(tok_table, pos_table, idx)


def kernel(inpTok, tok_table, pos_table):
    pos_rep = jnp.broadcast_to(pos_table, (NW, T, D))
    return _emb(tok_table, pos_rep, inpTok.astype(jnp.int32))


# speedup vs baseline: 1.7700x; 1.0708x over previous
import functools

import jax
import jax.numpy as jnp
from jax import lax
from jax.experimental import pallas as pl
from jax.experimental.pallas import tpu as pltpu
from jax.experimental.pallas import tpu_sc as plsc

VOC = 100000
D = 128
T = 100
B = 16384
NC = 2
NS = 16
NW = NC * NS
SEQ_PER_W = B // NW
S_CH = 2
N_CH = SEQ_PER_W // S_CH
NSLOT = 4


def _body(tok_hbm, pos_hbm, idx_hbm, out_hbm, idx_v, rows_v, pos_v, *sems):
    sem_ab = sems[0:NSLOT]
    sem_c = sems[NSLOT:2 * NSLOT]
    sem_d = sems[2 * NSLOT:3 * NSLOT]
    wid = lax.axis_index("s") * NC + lax.axis_index("c")
    wbase = wid * SEQ_PER_W

    pltpu.sync_copy(pos_hbm.at[wid], pos_v)

    def ab_copies(c, s):
        seq0 = wbase + c * S_CH
        ops = [pltpu.make_async_copy(
            idx_hbm.at[pl.ds(seq0, S_CH)], idx_v.at[s], sem_ab[s])]
        if s % 2 == 0:
            ops.append(pltpu.make_async_copy(
                pos_hbm.at[wid], rows_v.at[s].at[0], sem_ab[s]))
        return ops

    def add_pos(s, seqs):
        def row(r, carry):
            for j in seqs:
                for v in range(8):
                    sl = pl.ds(v * 16, 16)
                    rows_v[s, j, r, sl] = rows_v[s, j, r, sl] + pos_v[r, sl]
            return carry
        lax.fori_loop(0, T, row, 0)

    def c_copies(s):
        return [pltpu.make_async_copy(
            tok_hbm.at[idx_v.at[s].at[j]], rows_v.at[s].at[j], sem_c[s])
            for j in range(S_CH)]

    def d_copy(c, s):
        seq0 = wbase + c * S_CH
        return pltpu.make_async_copy(
            rows_v.at[s], out_hbm.at[pl.ds(seq0, S_CH)], sem_d[s])

    def group(g, carry):
        for k in range(NSLOT):
            c = g * NSLOT + k

            @pl.when(jnp.logical_and(c >= NSLOT, c < N_CH + NSLOT))
            def _():
                d_copy(c - NSLOT, k).wait()

            @pl.when(c < N_CH)
            def _():
                for op in ab_copies(c, k):
                    op.start()

            @pl.when(jnp.logical_and(c >= 1, c < N_CH + 1))
            def _():
                kk = (k - 1) % NSLOT
                for op in ab_copies(c - 1, kk):
                    op.wait()
                ops = c_copies(kk)
                if kk % 2 == 0:
                    ops[0].start(add=True)
                else:
                    ops[0].start()
                ops[1].start()

            @pl.when(jnp.logical_and(c >= 2, c < N_CH + 2))
            def _():
                kk = (k - 2) % NSLOT
                for op in c_copies(kk):
                    op.wait()
                add_pos(kk, (1,) if kk % 2 == 0 else (0, 1))
                d_copy(c - 2, kk).start()
        return carry

    lax.fori_loop(0, (N_CH + NSLOT) // NSLOT, group, 0)


@jax.jit
def _emb(tok_table, pos_table, idx):
    grid_kernel = pl.kernel(
        _body,
        out_type=jax.ShapeDtypeStruct((B, T, D), jnp.float32),
        mesh=plsc.VectorSubcoreMesh(
            core_axis_name="c", subcore_axis_name="s",
            num_cores=NC, num_subcores=NS),
        scratch_types=[
            pltpu.VMEM((NSLOT, S_CH, T), jnp.int32),
            pltpu.VMEM((NSLOT, S_CH, T, D), jnp.float32),
            pltpu.VMEM((T, D), jnp.float32),
        ] + [pltpu.SemaphoreType.DMA] * (3 * NSLOT),
    )
    return grid_kernel(tok_table, pos_table, idx)


def kernel(inpTok, tok_table, pos_table):
    pos_rep = jnp.broadcast_to(pos_table, (NW, T, D))
    return _emb(tok_table, pos_rep, inpTok.astype(jnp.int32))


# speedup vs baseline: 1.7743x; 1.0024x over previous
import jax
import jax.numpy as jnp
from jax import lax
from jax.experimental import pallas as pl
from jax.experimental.pallas import tpu as pltpu
from jax.experimental.pallas import tpu_sc as plsc

VOC = 100000
D = 128
T = 100
B = 16384
NC = 2
NS = 16
NW = NC * NS
SEQ_PER_W = B // NW
S_CH = 2
N_CH = SEQ_PER_W // S_CH
NSLOT = 4


def _body(tok_hbm, pos_hbm, idx_hbm, out_hbm, idx_v, rows_v, pos_v, *sems):
    sem_ab = sems[0:NSLOT]
    sem_c = sems[NSLOT:2 * NSLOT]
    sem_d = sems[2 * NSLOT:3 * NSLOT]
    wid = lax.axis_index("s") * NC + lax.axis_index("c")
    wbase = wid * SEQ_PER_W

    pltpu.sync_copy(pos_hbm.at[wid], pos_v)

    def ab_copies(c, s):
        seq0 = wbase + c * S_CH
        ops = [pltpu.make_async_copy(
            idx_hbm.at[pl.ds(seq0, S_CH)], idx_v.at[s], sem_ab[s])]
        if s % 2 == 0:
            ops.append(pltpu.make_async_copy(
                pos_hbm.at[wid], rows_v.at[s].at[0], sem_ab[s]))
        return ops

    def add_pos(s, seqs):
        def row(r, carry):
            for j in seqs:
                for v in range(8):
                    sl = pl.ds(v * 16, 16)
                    rows_v[s, j, r, sl] = rows_v[s, j, r, sl] + pos_v[r, sl]
            return carry
        lax.fori_loop(0, T, row, 0)

    def c_copies(s):
        return [pltpu.make_async_copy(
            tok_hbm.at[idx_v.at[s].at[j]], rows_v.at[s].at[j], sem_c[s])
            for j in range(S_CH)]

    def d_copy(c, s):
        seq0 = wbase + c * S_CH
        return pltpu.make_async_copy(
            rows_v.at[s], out_hbm.at[pl.ds(seq0, S_CH)], sem_d[s])

    def group(g, carry):
        for k in range(NSLOT):
            c = g * NSLOT + k

            @pl.when(jnp.logical_and(c >= NSLOT, c < N_CH + NSLOT))
            def _():
                d_copy(c - NSLOT, k).wait()

            @pl.when(c < N_CH)
            def _():
                for op in ab_copies(c, k):
                    op.start()

            @pl.when(jnp.logical_and(c >= 1, c < N_CH + 1))
            def _():
                kk = (k - 1) % NSLOT
                for op in ab_copies(c - 1, kk):
                    op.wait()
                ops = c_copies(kk)
                if kk % 2 == 0:
                    ops[0].start(add=True)
                else:
                    ops[0].start()
                ops[1].start()

            @pl.when(jnp.logical_and(c >= 2, c < N_CH + 2))
            def _():
                kk = (k - 2) % NSLOT
                for op in c_copies(kk):
                    op.wait()
                add_pos(kk, (1,) if kk % 2 == 0 else (0, 1))
                d_copy(c - 2, kk).start()
        return carry

    lax.fori_loop(0, (N_CH + NSLOT) // NSLOT, group, 0)


@jax.jit
def _emb(tok_table, pos_table, idx):
    grid_kernel = pl.kernel(
        _body,
        out_type=jax.ShapeDtypeStruct((B, T, D), jnp.float32),
        mesh=plsc.VectorSubcoreMesh(
            core_axis_name="c", subcore_axis_name="s",
            num_cores=NC, num_subcores=NS),
        scratch_types=[
            pltpu.VMEM((NSLOT, S_CH, T), jnp.int32),
            pltpu.VMEM((NSLOT, S_CH, T, D), jnp.float32),
            pltpu.VMEM((T, D), jnp.float32),
        ] + [pltpu.SemaphoreType.DMA] * (3 * NSLOT),
    )
    return grid_kernel(tok_table, pos_table, idx)


def kernel(inpTok, tok_table, pos_table):
    pos_rep = jnp.broadcast_to(pos_table, (NW, T, D))
    return _emb(tok_table, pos_rep, inpTok.astype(jnp.int32))
